# baseline (speedup 1.0000x reference)
import functools

import jax
import jax.numpy as jnp
from jax import lax
from jax.experimental import pallas as pl
from jax.experimental.pallas import tpu as pltpu
from jax.experimental.pallas import tpu_sc as plsc

_NC = 2
_NS = 16
_NW = _NC * _NS

_EPS = 1e-12


def _spmm_sc(x, src, dst, ew, npad, ch=40):
    d = x.shape[1]
    e = src.shape[0]
    epw = e // _NW
    nch = epw // ch
    npair = nch // 2
    rpt = npad // _NS

    mesh = plsc.VectorSubcoreMesh(core_axis_name="c", subcore_axis_name="s", num_cores=_NC, num_subcores=_NS)

    @functools.partial(
        pl.kernel,
        out_type=jax.ShapeDtypeStruct((_NC * npad, d), jnp.float32),
        mesh=mesh,
        scratch_types=[
            pltpu.VMEM_SHARED((npad, d), jnp.float32),
            pltpu.VMEM((epw,), jnp.int32),
            pltpu.VMEM((epw,), jnp.float32),
            pltpu.VMEM((ch,), jnp.int32),
            pltpu.VMEM((ch,), jnp.int32),
            pltpu.VMEM((ch, d), jnp.float32),
            pltpu.VMEM((ch, d), jnp.float32),
            pltpu.SemaphoreType.DMA,
            pltpu.SemaphoreType.DMA,
            pltpu.SemaphoreType.DMA,
            pltpu.SemaphoreType.DMA,
            pltpu.SemaphoreType.DMA,
            pltpu.SemaphoreType.DMA,
        ],
        compiler_params=pltpu.CompilerParams(use_tc_tiling_on_sc=True),
    )
    def k(x_hbm, src_hbm, dst_hbm, ew_hbm, out_hbm,
          acc, sidx_tab, ewv_tab, didx0, didx1, rows0, rows1,
          sg0, sg1, sd0, sd1, ss0, ss1):
        c = lax.axis_index("c")
        s = lax.axis_index("s")
        w = s * _NC + c
        ebase = w * epw
        rlo = s * rpt
        rows = (rows0, rows1)
        didx = (didx0, didx1)
        gsem = (sg0, sg1)
        dsem = (sd0, sd1)
        ssem = (ss0, ss1)

        pltpu.sync_copy(src_hbm.at[pl.ds(ebase, epw)], sidx_tab)
        pltpu.sync_copy(ew_hbm.at[pl.ds(ebase, epw)], ewv_tab)

        def zrow(r, cc):
            for j in range(d // 16):
                rows0[r, pl.ds(16 * j, 16)] = jnp.zeros((16,), jnp.float32)
            return cc

        lax.fori_loop(0, ch, zrow, 0)

        def zcp(z, cc):
            pltpu.sync_copy(rows0, acc.at[pl.ds(rlo + z * ch, ch)])
            return cc

        lax.fori_loop(0, rpt // ch, zcp, 0)
        plsc.subcore_barrier()

        def dfire(i, b):
            pltpu.async_copy(dst_hbm.at[pl.ds(ebase + i * ch, ch)], didx[b], dsem[b])

        def dwait(i, b):
            pltpu.make_async_copy(dst_hbm.at[pl.ds(ebase + i * ch, ch)],
                                  didx[b], dsem[b]).wait()

        def gfire(i, b):
            pltpu.async_copy(x_hbm.at[sidx_tab.at[pl.ds(i * ch, ch)]],
                             rows[b], gsem[b])

        def gwait(i, b):
            pltpu.make_async_copy(x_hbm.at[sidx_tab.at[pl.ds(i * ch, ch)]],
                                  rows[b], gsem[b]).wait()

        def sfire(b):
            pltpu.async_copy(rows[b], acc.at[didx[b]], ssem[b], add=True)

        def swait(b):
            pltpu.make_async_copy(rows[b], acc.at[didx[b]], ssem[b]).wait()

        def scale(i, b):
            rb = rows[b]

            def grp(g, cc):
                wv = ewv_tab[pl.ds(i * ch + g * 16, 16)]
                for lane in range(16):
                    wgt = wv[lane]
                    r = g * 16 + lane
                    for j in range(d // 16):
                        sl = pl.ds(j * 16, 16)
                        rb[r, sl] = rb[r, sl] * wgt
                return cc

            lax.fori_loop(0, ch // 16, grp, 0)
            if ch % 16:
                wv = ewv_tab[pl.ds(i * ch + ch - 16, 16)]
                for lane in range(16 - ch % 16, 16):
                    wgt = wv[lane]
                    r = ch - 16 + lane
                    for j in range(d // 16):
                        sl = pl.ds(j * 16, 16)
                        rb[r, sl] = rb[r, sl] * wgt

        dfire(0, 0)
        gfire(0, 0)

        def pair(p, cc):
            i0 = 2 * p
            i1 = i0 + 1
            dfire(i1, 1)
            gfire(i1, 1)
            gwait(i0, 0)
            scale(i0, 0)
            dwait(i0, 0)
            sfire(0)
            gwait(i1, 1)
            swait(0)

            @pl.when(p + 1 < npair)
            def _():
                dfire(i0 + 2, 0)
                gfire(i0 + 2, 0)

            scale(i1, 1)
            dwait(i1, 1)
            sfire(1)
            swait(1)
            return cc

        lax.fori_loop(0, npair, pair, 0)
        plsc.subcore_barrier()
        pltpu.sync_copy(acc.at[pl.ds(rlo, rpt)],
                        out_hbm.at[pl.ds(c * npad + rlo, rpt)])

    return k(x, src, dst, ew)


def _gather_sc(table, idx):
    n, d4 = table.shape
    m = idx.shape[0]
    per_w = m // _NW
    ch = 128
    nstep = per_w // ch

    mesh = plsc.VectorSubcoreMesh(core_axis_name="c", subcore_axis_name="s", num_cores=_NC, num_subcores=_NS)

    @functools.partial(
        pl.kernel,
        out_type=jax.ShapeDtypeStruct((m, d4), jnp.float32),
        mesh=mesh,
        scratch_types=[
            pltpu.VMEM((ch,), jnp.int32),
            pltpu.VMEM((ch, d4), jnp.float32),
            pltpu.SemaphoreType.DMA,
        ],
    )
    def k(tab_hbm, idx_hbm, out_hbm, ib, rows, sem):
        c = lax.axis_index("c")
        s = lax.axis_index("s")
        wid = s * _NC + c
        for t in range(nstep):
            base = wid * per_w + t * ch
            pltpu.sync_copy(idx_hbm.at[pl.ds(base, ch)], ib)
            pltpu.async_copy(tab_hbm.at[ib], rows, sem).wait()
            pltpu.sync_copy(rows, out_hbm.at[pl.ds(base, ch)])

    return k(table, idx)


def _proj_body(img_ref, rel_ref, att_ref, wi_ref, wr_ref, wa_ref,
               bi_ref, br_ref, ba_ref, oi_ref, or_ref, oa_ref):
    x = img_ref[...]
    nrm = jnp.sqrt(jnp.sum(x * x, axis=1, keepdims=True))
    x = x / jnp.maximum(nrm, _EPS)
    dn = (((1,), (1,)), ((), ()))
    oi_ref[...] = lax.dot_general(x, wi_ref[...], dn,
                                  preferred_element_type=jnp.float32) + bi_ref[...]
    or_ref[...] = lax.dot_general(rel_ref[...], wr_ref[...], dn,
                                  preferred_element_type=jnp.float32) + br_ref[...]
    oa_ref[...] = lax.dot_general(att_ref[...], wa_ref[...], dn,
                                  preferred_element_type=jnp.float32) + ba_ref[...]


def _proj(img, rel_p, att_p, w_img, w_rel_p, w_att_p, b_img, b_rel, b_att):
    n = img.shape[0]
    bm = 1000
    g = n // bm
    d = w_img.shape[0]
    kimg, krel = img.shape[1], rel_p.shape[1]
    out = jax.ShapeDtypeStruct((n, d), jnp.float32)
    full = lambda shp: pl.BlockSpec(shp, lambda i: tuple(0 for _ in shp))
    return pl.pallas_call(
        _proj_body,
        grid=(g,),
        in_specs=[
            pl.BlockSpec((bm, kimg), lambda i: (i, 0)),
            pl.BlockSpec((bm, krel), lambda i: (i, 0)),
            pl.BlockSpec((bm, krel), lambda i: (i, 0)),
            full((d, kimg)), full((d, krel)), full((d, krel)),
            full((d,)), full((d,)), full((d,)),
        ],
        out_specs=[pl.BlockSpec((bm, d), lambda i: (i, 0))] * 3,
        out_shape=[out, out, out],
    )(img, rel_p, att_p, w_img, w_rel_p, w_att_p, b_img, b_rel, b_att)


def _affine2(ax2, w, b, relu):
    twon, d = ax2.shape
    n = twon // _NC
    bm = 1024
    g = n // bm

    def body(a_ref, b2_ref, w_ref, bias_ref, o_ref):
        p = a_ref[...] + b2_ref[...]
        z = jnp.dot(p, w_ref[...], preferred_element_type=jnp.float32) + bias_ref[...]
        o_ref[...] = jnp.maximum(z, 0.0) if relu else z

    return pl.pallas_call(
        body,
        grid=(g,),
        in_specs=[
            pl.BlockSpec((bm, d), lambda i: (i, 0)),
            pl.BlockSpec((bm, d), lambda i: (g + i, 0)),
            pl.BlockSpec((d, d), lambda i: (0, 0)),
            pl.BlockSpec((d,), lambda i: (0,)),
        ],
        out_specs=pl.BlockSpec((bm, d), lambda i: (i, 0)),
        out_shape=jax.ShapeDtypeStruct((n, d), jnp.float32),
    )(ax2, ax2, w, b)


def _joint(img_emb, att_emb, rel_emb, gph_emb, wn):
    n, d = img_emb.shape
    bm = 1000
    g = n // bm

    def body(wn_ref, i_ref, a_ref, r_ref, g_ref, o_ref):
        for k, ref in enumerate((i_ref, a_ref, r_ref, g_ref)):
            x = ref[...]
            nrm = jnp.sqrt(jnp.sum(x * x, axis=1, keepdims=True))
            o_ref[:, k * d:(k + 1) * d] = wn_ref[k] * (x / jnp.maximum(nrm, _EPS))

    return pl.pallas_call(
        body,
        grid=(g,),
        in_specs=[
            pl.BlockSpec(memory_space=pltpu.SMEM),
        ] + [pl.BlockSpec((bm, d), lambda i: (i, 0))] * 4,
        out_specs=pl.BlockSpec((bm, 4 * d), lambda i: (i, 0)),
        out_shape=jax.ShapeDtypeStruct((n, 4 * d), jnp.float32),
    )(wn, img_emb, att_emb, rel_emb, gph_emb)


_ALPHAS = (5.0, 15.0, 15.0, 15.0, 15.0)
_GRP = (3, 2, 1, 0)


def _scores_body(scal_ref, a_ref, b_ref, *outs):
    i = pl.program_id(0)
    j = pl.program_id(1)
    bm = a_ref.shape[0]
    d = a_ref.shape[1] // 4
    a = a_ref[...]
    b = b_ref[...]
    dn = (((1,), (1,)), ((), ()))
    gmats = [lax.dot_general(a[:, k * d:(k + 1) * d], b[:, k * d:(k + 1) * d],
                             dn, preferred_element_type=jnp.float32)
             for k in range(4)]
    sjoint = (gmats[0] + gmats[1] + gmats[2] + gmats[3]) * scal_ref[4]
    svals = [gmats[_GRP[l]] * scal_ref[_GRP[l]] for l in range(4)] + [sjoint]

    rr = lax.broadcasted_iota(jnp.int32, (bm, bm), 0)
    cc = lax.broadcasted_iota(jnp.int32, (bm, bm), 1)
    offdiag = jnp.where((rr == cc) & (i == j), 0.0, 1.0)

    rs_refs = outs[:5]
    cp_refs = outs[5:]
    for l in range(5):
        el = jnp.exp(_ALPHAS[l] * svals[l]) * offdiag
        rsum = jnp.sum(el, axis=1)
        csum = jnp.sum(el, axis=0)

        @pl.when(j == 0)
        def _(rref=rs_refs[l], v=rsum):
            rref[...] = v

        @pl.when(j != 0)
        def _(rref=rs_refs[l], v=rsum):
            rref[...] = rref[...] + v

        cp_refs[l][...] = csum.reshape(1, 1, bm)


def _scores(jg, scal):
    m2, d4 = jg.shape
    m = m2 // 2
    bm = 512
    g = m // bm
    rs_shape = jax.ShapeDtypeStruct((m,), jnp.float32)
    cp_shape = jax.ShapeDtypeStruct((g, 1, m), jnp.float32)
    return pl.pallas_call(
        _scores_body,
        grid=(g, g),
        in_specs=[
            pl.BlockSpec(memory_space=pltpu.SMEM),
            pl.BlockSpec((bm, d4), lambda i, j: (i, 0)),
            pl.BlockSpec((bm, d4), lambda i, j: (g + j, 0)),
        ],
        out_specs=([pl.BlockSpec((bm,), lambda i, j: (i,))] * 5
                   + [pl.BlockSpec((1, 1, bm), lambda i, j: (i, 0, j))] * 5),
        out_shape=[rs_shape] * 5 + [cp_shape] * 5,
    )(scal, jg, jg)


def _finish_body(scal_ref, ai_ref, as_ref, *refs):
    i = pl.program_id(0)
    rs_refs = refs[:5]
    cp_refs = refs[5:10]
    o_ref = refs[10]
    bm = ai_ref.shape[0]
    d = ai_ref.shape[1] // 4
    ai = ai_ref[...]
    asv = as_ref[...]
    gd = [jnp.sum(ai[:, k * d:(k + 1) * d] * asv[:, k * d:(k + 1) * d], axis=1)
          for k in range(4)]
    djoint = (gd[0] + gd[1] + gd[2] + gd[3]) * scal_ref[4]
    dvals = [gd[_GRP[l]] * scal_ref[_GRP[l]] for l in range(4)] + [djoint]

    total = jnp.zeros((), jnp.float32)
    for l in range(5):
        rs = rs_refs[l][...]
        cs = jnp.sum(cp_refs[l][...], axis=(0, 1))
        dg = dvals[l]
        term = (jnp.log1p(cs) / _ALPHAS[l]
                + jnp.log1p(rs) / _ALPHAS[l]
                - 10.0 * jnp.log1p(jnp.maximum(dg, 0.0)))
        total = total + jnp.sum(term)

    @pl.when(i == 0)
    def _():
        o_ref[...] = jnp.zeros((1, 1), jnp.float32)

    o_ref[...] = o_ref[...] + total.reshape(1, 1) * (1.0 / (bm * pl.num_programs(0)))


def _finish(jg, rs_list, cp_list, scal):
    m2, d4 = jg.shape
    m = m2 // 2
    bm = 512
    g = m // bm
    return pl.pallas_call(
        _finish_body,
        grid=(g,),
        in_specs=(
            [pl.BlockSpec(memory_space=pltpu.SMEM),
             pl.BlockSpec((bm, d4), lambda i: (i, 0)),
             pl.BlockSpec((bm, d4), lambda i: (g + i, 0))]
            + [pl.BlockSpec((bm,), lambda i: (i,))] * 5
            + [pl.BlockSpec((g, 1, bm), lambda i: (0, 0, i))] * 5
        ),
        out_specs=pl.BlockSpec((1, 1), lambda i: (0, 0)),
        out_shape=jax.ShapeDtypeStruct((1, 1), jnp.float32),
    )(scal, jg, jg, *rs_list, *cp_list)


def kernel(batch, src, dst, edge_weight, ent_weight, img_feats, rel_feats,
           att_feats, gcn_w1, gcn_b1, gcn_w2, gcn_b2, w_img, b_img, w_rel,
           b_rel, w_att, b_att, weight_raw):
    n, d = ent_weight.shape

    rel_p = jnp.pad(rel_feats, ((0, 0), (0, 24)))
    att_p = jnp.pad(att_feats, ((0, 0), (0, 24)))
    wrel_p = jnp.pad(w_rel, ((0, 0), (0, 24)))
    watt_p = jnp.pad(w_att, ((0, 0), (0, 24)))

    img_emb, rel_emb, att_emb = _proj(img_feats, rel_p, att_p,
                                      w_img, wrel_p, watt_p,
                                      b_img, b_rel, b_att)

    npad = 10240
    src32 = src.astype(jnp.int32)
    dst32 = dst.astype(jnp.int32)
    ax = _spmm_sc(ent_weight, src32, dst32, edge_weight, npad)
    h = _affine2(ax, gcn_w1, gcn_b1, relu=True)
    ah = _spmm_sc(h, src32, dst32, edge_weight, npad)
    gph_emb = _affine2(ah, gcn_w2, gcn_b2, relu=False)[:n]

    wn = jax.nn.softmax(weight_raw)
    joint = _joint(img_emb, att_emb, rel_emb, gph_emb, wn)

    bt = jnp.transpose(batch).reshape(-1).astype(jnp.int32)
    jg = _gather_sc(joint, bt)

    w2 = wn * wn
    scal = jnp.concatenate(
        [1.0 / w2, (1.0 / jnp.sum(w2))[None], jnp.zeros((3,), jnp.float32)]
    ).astype(jnp.float32)

    outs = _scores(jg, scal)
    loss = _finish(jg, outs[:5], outs[5:], scal)
    return loss.reshape(()), joint


# speedup vs baseline: 1.4561x; 1.4561x over previous
import functools

import jax
import jax.numpy as jnp
from jax import lax
from jax.experimental import pallas as pl
from jax.experimental.pallas import tpu as pltpu
from jax.experimental.pallas import tpu_sc as plsc

_NC = 2
_NS = 16
_NW = _NC * _NS

_EPS = 1e-12


def _spmm_sc(x, src, dst, ew, npad, ch=40):
    d = x.shape[1]
    e = src.shape[0]
    epw = e // _NW
    nch = epw // ch
    npair = nch // 2
    rpt = npad // _NS

    mesh = plsc.VectorSubcoreMesh(core_axis_name="c", subcore_axis_name="s", num_cores=_NC, num_subcores=_NS)

    @functools.partial(
        pl.kernel,
        out_type=jax.ShapeDtypeStruct((_NC * npad, d), jnp.float32),
        mesh=mesh,
        scratch_types=[
            pltpu.VMEM_SHARED((npad, d), jnp.float32),
            pltpu.VMEM((epw,), jnp.int32),
            pltpu.VMEM((epw,), jnp.float32),
            pltpu.VMEM((ch,), jnp.int32),
            pltpu.VMEM((ch,), jnp.int32),
            pltpu.VMEM((ch, d), jnp.float32),
            pltpu.VMEM((ch, d), jnp.float32),
            pltpu.SemaphoreType.DMA,
            pltpu.SemaphoreType.DMA,
            pltpu.SemaphoreType.DMA,
            pltpu.SemaphoreType.DMA,
            pltpu.SemaphoreType.DMA,
            pltpu.SemaphoreType.DMA,
        ],
        compiler_params=pltpu.CompilerParams(use_tc_tiling_on_sc=True),
    )
    def k(x_hbm, src_hbm, dst_hbm, ew_hbm, out_hbm,
          acc, sidx_tab, ewv_tab, didx0, didx1, rows0, rows1,
          sg0, sg1, sd0, sd1, ss0, ss1):
        c = lax.axis_index("c")
        s = lax.axis_index("s")
        w = s * _NC + c
        ebase = w * epw
        rlo = s * rpt
        rows = (rows0, rows1)
        didx = (didx0, didx1)
        gsem = (sg0, sg1)
        dsem = (sd0, sd1)
        ssem = (ss0, ss1)

        pltpu.sync_copy(src_hbm.at[pl.ds(ebase, epw)], sidx_tab)
        pltpu.sync_copy(ew_hbm.at[pl.ds(ebase, epw)], ewv_tab)

        def zrow(r, cc):
            for j in range(d // 16):
                rows0[r, pl.ds(16 * j, 16)] = jnp.zeros((16,), jnp.float32)
            return cc

        lax.fori_loop(0, ch, zrow, 0)

        def zcp(z, cc):
            pltpu.sync_copy(rows0, acc.at[pl.ds(rlo + z * ch, ch)])
            return cc

        lax.fori_loop(0, rpt // ch, zcp, 0)
        plsc.subcore_barrier()

        def dfire(i, b):
            pltpu.async_copy(dst_hbm.at[pl.ds(ebase + i * ch, ch)], didx[b], dsem[b])

        def dwait(i, b):
            pltpu.make_async_copy(dst_hbm.at[pl.ds(ebase + i * ch, ch)],
                                  didx[b], dsem[b]).wait()

        def gfire(i, b):
            pltpu.async_copy(x_hbm.at[sidx_tab.at[pl.ds(i * ch, ch)]],
                             rows[b], gsem[b])

        def gwait(i, b):
            pltpu.make_async_copy(x_hbm.at[sidx_tab.at[pl.ds(i * ch, ch)]],
                                  rows[b], gsem[b]).wait()

        def sfire(b):
            pltpu.async_copy(rows[b], acc.at[didx[b]], ssem[b], add=True)

        def swait(b):
            pltpu.make_async_copy(rows[b], acc.at[didx[b]], ssem[b]).wait()

        def scale(i, b):
            rb = rows[b]

            def grp(g, cc):
                wv = ewv_tab[pl.ds(i * ch + g * 16, 16)]
                for lane in range(16):
                    wgt = wv[lane]
                    r = g * 16 + lane
                    for j in range(d // 16):
                        sl = pl.ds(j * 16, 16)
                        rb[r, sl] = rb[r, sl] * wgt
                return cc

            lax.fori_loop(0, ch // 16, grp, 0)
            if ch % 16:
                wv = ewv_tab[pl.ds(i * ch + ch - 16, 16)]
                for lane in range(16 - ch % 16, 16):
                    wgt = wv[lane]
                    r = ch - 16 + lane
                    for j in range(d // 16):
                        sl = pl.ds(j * 16, 16)
                        rb[r, sl] = rb[r, sl] * wgt

        dfire(0, 0)
        gfire(0, 0)

        def pair(p, cc):
            i0 = 2 * p
            i1 = i0 + 1
            dfire(i1, 1)
            gfire(i1, 1)
            gwait(i0, 0)
            scale(i0, 0)
            dwait(i0, 0)
            sfire(0)
            gwait(i1, 1)
            swait(0)

            @pl.when(p + 1 < npair)
            def _():
                dfire(i0 + 2, 0)
                gfire(i0 + 2, 0)

            scale(i1, 1)
            dwait(i1, 1)
            sfire(1)
            swait(1)
            return cc

        lax.fori_loop(0, npair, pair, 0)
        plsc.subcore_barrier()
        pltpu.sync_copy(acc.at[pl.ds(rlo, rpt)],
                        out_hbm.at[pl.ds(c * npad + rlo, rpt)])

    return k(x, src, dst, ew)


def _gather_sc(table, idx):
    n, d4 = table.shape
    m = idx.shape[0]
    per_w = m // _NW
    ch = 128
    nstep = per_w // ch

    mesh = plsc.VectorSubcoreMesh(core_axis_name="c", subcore_axis_name="s", num_cores=_NC, num_subcores=_NS)

    @functools.partial(
        pl.kernel,
        out_type=jax.ShapeDtypeStruct((m, d4), jnp.float32),
        mesh=mesh,
        scratch_types=[
            pltpu.VMEM((ch,), jnp.int32),
            pltpu.VMEM((ch, d4), jnp.float32),
            pltpu.SemaphoreType.DMA,
        ],
    )
    def k(tab_hbm, idx_hbm, out_hbm, ib, rows, sem):
        c = lax.axis_index("c")
        s = lax.axis_index("s")
        wid = s * _NC + c
        for t in range(nstep):
            base = wid * per_w + t * ch
            pltpu.sync_copy(idx_hbm.at[pl.ds(base, ch)], ib)
            pltpu.async_copy(tab_hbm.at[ib], rows, sem).wait()
            pltpu.sync_copy(rows, out_hbm.at[pl.ds(base, ch)])

    return k(table, idx)


def _proj_body(img_ref, rel_ref, att_ref, wi_ref, wr_ref, wa_ref,
               bi_ref, br_ref, ba_ref, oi_ref, or_ref, oa_ref):
    x = img_ref[...]
    nrm = jnp.sqrt(jnp.sum(x * x, axis=1, keepdims=True))
    x = x / jnp.maximum(nrm, _EPS)
    dn = (((1,), (1,)), ((), ()))
    oi_ref[...] = lax.dot_general(x, wi_ref[...], dn,
                                  preferred_element_type=jnp.float32) + bi_ref[...]
    or_ref[...] = lax.dot_general(rel_ref[...], wr_ref[...], dn,
                                  preferred_element_type=jnp.float32) + br_ref[...]
    oa_ref[...] = lax.dot_general(att_ref[...], wa_ref[...], dn,
                                  preferred_element_type=jnp.float32) + ba_ref[...]


def _proj(img, rel_p, att_p, w_img, w_rel_p, w_att_p, b_img, b_rel, b_att):
    n = img.shape[0]
    bm = 1000
    g = n // bm
    d = w_img.shape[0]
    kimg, krel = img.shape[1], rel_p.shape[1]
    out = jax.ShapeDtypeStruct((n, d), jnp.float32)
    full = lambda shp: pl.BlockSpec(shp, lambda i: tuple(0 for _ in shp))
    return pl.pallas_call(
        _proj_body,
        grid=(g,),
        in_specs=[
            pl.BlockSpec((bm, kimg), lambda i: (i, 0)),
            pl.BlockSpec((bm, krel), lambda i: (i, 0)),
            pl.BlockSpec((bm, krel), lambda i: (i, 0)),
            full((d, kimg)), full((d, krel)), full((d, krel)),
            full((d,)), full((d,)), full((d,)),
        ],
        out_specs=[pl.BlockSpec((bm, d), lambda i: (i, 0))] * 3,
        out_shape=[out, out, out],
    )(img, rel_p, att_p, w_img, w_rel_p, w_att_p, b_img, b_rel, b_att)


def _affine2(ax2, w, b, relu):
    twon, d = ax2.shape
    n = twon // _NC
    bm = 1024
    g = n // bm

    def body(a_ref, b2_ref, w_ref, bias_ref, o_ref):
        p = a_ref[...] + b2_ref[...]
        z = jnp.dot(p, w_ref[...], preferred_element_type=jnp.float32) + bias_ref[...]
        o_ref[...] = jnp.maximum(z, 0.0) if relu else z

    return pl.pallas_call(
        body,
        grid=(g,),
        in_specs=[
            pl.BlockSpec((bm, d), lambda i: (i, 0)),
            pl.BlockSpec((bm, d), lambda i: (g + i, 0)),
            pl.BlockSpec((d, d), lambda i: (0, 0)),
            pl.BlockSpec((d,), lambda i: (0,)),
        ],
        out_specs=pl.BlockSpec((bm, d), lambda i: (i, 0)),
        out_shape=jax.ShapeDtypeStruct((n, d), jnp.float32),
    )(ax2, ax2, w, b)


def _joint(img_emb, att_emb, rel_emb, gph_emb, wn):
    n, d = img_emb.shape
    bm = 1000
    g = n // bm

    def body(wn_ref, i_ref, a_ref, r_ref, g_ref, o_ref):
        for k, ref in enumerate((i_ref, a_ref, r_ref, g_ref)):
            x = ref[...]
            nrm = jnp.sqrt(jnp.sum(x * x, axis=1, keepdims=True))
            o_ref[:, k * d:(k + 1) * d] = wn_ref[k] * (x / jnp.maximum(nrm, _EPS))

    return pl.pallas_call(
        body,
        grid=(g,),
        in_specs=[
            pl.BlockSpec(memory_space=pltpu.SMEM),
        ] + [pl.BlockSpec((bm, d), lambda i: (i, 0))] * 4,
        out_specs=pl.BlockSpec((bm, 4 * d), lambda i: (i, 0)),
        out_shape=jax.ShapeDtypeStruct((n, 4 * d), jnp.float32),
    )(wn, img_emb, att_emb, rel_emb, gph_emb)


_ALPHAS = (5.0, 15.0, 15.0, 15.0, 15.0)
_GRP = (3, 2, 1, 0)


def _scores_body(scal_ref, a_ref, b_ref, *outs):
    i = pl.program_id(0)
    j = pl.program_id(1)
    bm = a_ref.shape[0]
    d = a_ref.shape[1] // 4
    a = a_ref[...]
    b = b_ref[...]
    dn = (((1,), (1,)), ((), ()))
    gmats = [lax.dot_general(a[:, k * d:(k + 1) * d], b[:, k * d:(k + 1) * d],
                             dn, preferred_element_type=jnp.float32)
             for k in range(4)]
    sjoint = (gmats[0] + gmats[1] + gmats[2] + gmats[3]) * scal_ref[4]
    svals = [gmats[_GRP[l]] * scal_ref[_GRP[l]] for l in range(4)] + [sjoint]

    rr = lax.broadcasted_iota(jnp.int32, (bm, bm), 0)
    cc = lax.broadcasted_iota(jnp.int32, (bm, bm), 1)
    offdiag = jnp.where((rr == cc) & (i == j), 0.0, 1.0)

    rs_refs = outs[:5]
    cp_refs = outs[5:]
    for l in range(5):
        el = jnp.exp(_ALPHAS[l] * svals[l]) * offdiag
        rsum = jnp.sum(el, axis=1)
        csum = jnp.sum(el, axis=0)

        @pl.when(j == 0)
        def _(rref=rs_refs[l], v=rsum):
            rref[...] = v

        @pl.when(j != 0)
        def _(rref=rs_refs[l], v=rsum):
            rref[...] = rref[...] + v

        cp_refs[l][...] = csum.reshape(1, 1, bm)


def _scores(jg, scal):
    m2, d4 = jg.shape
    m = m2 // 2
    bm = 512
    g = m // bm
    rs_shape = jax.ShapeDtypeStruct((m,), jnp.float32)
    cp_shape = jax.ShapeDtypeStruct((g, 1, m), jnp.float32)
    return pl.pallas_call(
        _scores_body,
        grid=(g, g),
        in_specs=[
            pl.BlockSpec(memory_space=pltpu.SMEM),
            pl.BlockSpec((bm, d4), lambda i, j: (i, 0)),
            pl.BlockSpec((bm, d4), lambda i, j: (g + j, 0)),
        ],
        out_specs=([pl.BlockSpec((bm,), lambda i, j: (i,))] * 5
                   + [pl.BlockSpec((1, 1, bm), lambda i, j: (i, 0, j))] * 5),
        out_shape=[rs_shape] * 5 + [cp_shape] * 5,
    )(scal, jg, jg)


def _finish_body(scal_ref, ai_ref, as_ref, *refs):
    i = pl.program_id(0)
    rs_refs = refs[:5]
    cp_refs = refs[5:10]
    o_ref = refs[10]
    bm = ai_ref.shape[0]
    d = ai_ref.shape[1] // 4
    ai = ai_ref[...]
    asv = as_ref[...]
    gd = [jnp.sum(ai[:, k * d:(k + 1) * d] * asv[:, k * d:(k + 1) * d], axis=1)
          for k in range(4)]
    djoint = (gd[0] + gd[1] + gd[2] + gd[3]) * scal_ref[4]
    dvals = [gd[_GRP[l]] * scal_ref[_GRP[l]] for l in range(4)] + [djoint]

    total = jnp.zeros((), jnp.float32)
    for l in range(5):
        rs = rs_refs[l][...]
        cs = jnp.sum(cp_refs[l][...], axis=(0, 1))
        dg = dvals[l]
        term = (jnp.log1p(cs) / _ALPHAS[l]
                + jnp.log1p(rs) / _ALPHAS[l]
                - 10.0 * jnp.log1p(jnp.maximum(dg, 0.0)))
        total = total + jnp.sum(term)

    @pl.when(i == 0)
    def _():
        o_ref[...] = jnp.zeros((1, 1), jnp.float32)

    o_ref[...] = o_ref[...] + total.reshape(1, 1) * (1.0 / (bm * pl.num_programs(0)))


def _finish(jg, rs_list, cp_list, scal):
    m2, d4 = jg.shape
    m = m2 // 2
    bm = 512
    g = m // bm
    return pl.pallas_call(
        _finish_body,
        grid=(g,),
        in_specs=(
            [pl.BlockSpec(memory_space=pltpu.SMEM),
             pl.BlockSpec((bm, d4), lambda i: (i, 0)),
             pl.BlockSpec((bm, d4), lambda i: (g + i, 0))]
            + [pl.BlockSpec((bm,), lambda i: (i,))] * 5
            + [pl.BlockSpec((g, 1, bm), lambda i: (0, 0, i))] * 5
        ),
        out_specs=pl.BlockSpec((1, 1), lambda i: (0, 0)),
        out_shape=jax.ShapeDtypeStruct((1, 1), jnp.float32),
    )(scal, jg, jg, *rs_list, *cp_list)


def kernel(batch, src, dst, edge_weight, ent_weight, img_feats, rel_feats,
           att_feats, gcn_w1, gcn_b1, gcn_w2, gcn_b2, w_img, b_img, w_rel,
           b_rel, w_att, b_att, weight_raw):
    n, d = ent_weight.shape

    img_emb, rel_emb, att_emb = _proj(img_feats, rel_feats, att_feats,
                                      w_img, w_rel, w_att,
                                      b_img, b_rel, b_att)

    npad = 10240
    src32 = src.astype(jnp.int32)
    dst32 = dst.astype(jnp.int32)
    ax = _spmm_sc(ent_weight, src32, dst32, edge_weight, npad)
    h = _affine2(ax, gcn_w1, gcn_b1, relu=True)
    ah = _spmm_sc(h, src32, dst32, edge_weight, npad)
    gph_emb = _affine2(ah, gcn_w2, gcn_b2, relu=False)[:n]

    wn = jax.nn.softmax(weight_raw)
    joint = _joint(img_emb, att_emb, rel_emb, gph_emb, wn)

    bt = jnp.transpose(batch).reshape(-1).astype(jnp.int32)
    jg = _gather_sc(joint, bt)

    w2 = wn * wn
    scal = jnp.concatenate(
        [1.0 / w2, (1.0 / jnp.sum(w2))[None], jnp.zeros((3,), jnp.float32)]
    ).astype(jnp.float32)

    outs = _scores(jg, scal)
    loss = _finish(jg, outs[:5], outs[5:], scal)
    return loss.reshape(()), joint


# speedup vs baseline: 1.5209x; 1.0445x over previous
import functools

import jax
import jax.numpy as jnp
from jax import lax
from jax.experimental import pallas as pl
from jax.experimental.pallas import tpu as pltpu
from jax.experimental.pallas import tpu_sc as plsc

_NC = 2
_NS = 16
_NW = _NC * _NS

_EPS = 1e-12


def _spmm_sc(x, src, dst, ew, npad, ch=80):
    d = x.shape[1]
    e = src.shape[0]
    epw = e // _NW
    nch = epw // ch
    npair = nch // 2
    rpt = npad // _NS

    mesh = plsc.VectorSubcoreMesh(core_axis_name="c", subcore_axis_name="s", num_cores=_NC, num_subcores=_NS)

    @functools.partial(
        pl.kernel,
        out_type=jax.ShapeDtypeStruct((_NC * npad, d), jnp.float32),
        mesh=mesh,
        scratch_types=[
            pltpu.VMEM_SHARED((npad, d), jnp.float32),
            pltpu.VMEM((epw,), jnp.int32),
            pltpu.VMEM((epw,), jnp.float32),
            pltpu.VMEM((ch,), jnp.int32),
            pltpu.VMEM((ch,), jnp.int32),
            pltpu.VMEM((ch, d), jnp.float32),
            pltpu.VMEM((ch, d), jnp.float32),
            pltpu.SemaphoreType.DMA,
            pltpu.SemaphoreType.DMA,
            pltpu.SemaphoreType.DMA,
            pltpu.SemaphoreType.DMA,
            pltpu.SemaphoreType.DMA,
            pltpu.SemaphoreType.DMA,
        ],
        compiler_params=pltpu.CompilerParams(use_tc_tiling_on_sc=True),
    )
    def k(x_hbm, src_hbm, dst_hbm, ew_hbm, out_hbm,
          acc, sidx_tab, ewv_tab, didx0, didx1, rows0, rows1,
          sg0, sg1, sd0, sd1, ss0, ss1):
        c = lax.axis_index("c")
        s = lax.axis_index("s")
        w = s * _NC + c
        ebase = w * epw
        rlo = s * rpt
        rows = (rows0, rows1)
        didx = (didx0, didx1)
        gsem = (sg0, sg1)
        dsem = (sd0, sd1)
        ssem = (ss0, ss1)

        pltpu.sync_copy(src_hbm.at[pl.ds(ebase, epw)], sidx_tab)
        pltpu.sync_copy(ew_hbm.at[pl.ds(ebase, epw)], ewv_tab)

        def zrow(r, cc):
            for j in range(d // 16):
                rows0[r, pl.ds(16 * j, 16)] = jnp.zeros((16,), jnp.float32)
            return cc

        lax.fori_loop(0, ch, zrow, 0)

        def zcp(z, cc):
            pltpu.sync_copy(rows0, acc.at[pl.ds(rlo + z * ch, ch)])
            return cc

        lax.fori_loop(0, rpt // ch, zcp, 0)
        plsc.subcore_barrier()

        def dfire(i, b):
            pltpu.async_copy(dst_hbm.at[pl.ds(ebase + i * ch, ch)], didx[b], dsem[b])

        def dwait(i, b):
            pltpu.make_async_copy(dst_hbm.at[pl.ds(ebase + i * ch, ch)],
                                  didx[b], dsem[b]).wait()

        def gfire(i, b):
            pltpu.async_copy(x_hbm.at[sidx_tab.at[pl.ds(i * ch, ch)]],
                             rows[b], gsem[b])

        def gwait(i, b):
            pltpu.make_async_copy(x_hbm.at[sidx_tab.at[pl.ds(i * ch, ch)]],
                                  rows[b], gsem[b]).wait()

        def sfire(b):
            pltpu.async_copy(rows[b], acc.at[didx[b]], ssem[b], add=True)

        def swait(b):
            pltpu.make_async_copy(rows[b], acc.at[didx[b]], ssem[b]).wait()

        def scale(i, b):
            rb = rows[b]

            def grp(g, cc):
                wv = ewv_tab[pl.ds(i * ch + g * 16, 16)]
                for lane in range(16):
                    wgt = wv[lane]
                    r = g * 16 + lane
                    for j in range(d // 16):
                        sl = pl.ds(j * 16, 16)
                        rb[r, sl] = rb[r, sl] * wgt
                return cc

            lax.fori_loop(0, ch // 16, grp, 0)
            if ch % 16:
                wv = ewv_tab[pl.ds(i * ch + ch - 16, 16)]
                for lane in range(16 - ch % 16, 16):
                    wgt = wv[lane]
                    r = ch - 16 + lane
                    for j in range(d // 16):
                        sl = pl.ds(j * 16, 16)
                        rb[r, sl] = rb[r, sl] * wgt

        dfire(0, 0)
        gfire(0, 0)

        def pair(p, cc):
            i0 = 2 * p
            i1 = i0 + 1
            dfire(i1, 1)
            gfire(i1, 1)
            gwait(i0, 0)
            scale(i0, 0)
            dwait(i0, 0)
            sfire(0)
            gwait(i1, 1)
            swait(0)

            @pl.when(p + 1 < npair)
            def _():
                dfire(i0 + 2, 0)
                gfire(i0 + 2, 0)

            scale(i1, 1)
            dwait(i1, 1)
            sfire(1)
            swait(1)
            return cc

        lax.fori_loop(0, npair, pair, 0)
        if nch % 2:
            i_last = nch - 1
            dfire(i_last, 0)
            gfire(i_last, 0)
            gwait(i_last, 0)
            scale(i_last, 0)
            dwait(i_last, 0)
            sfire(0)
            swait(0)
        plsc.subcore_barrier()
        pltpu.sync_copy(acc.at[pl.ds(rlo, rpt)],
                        out_hbm.at[pl.ds(c * npad + rlo, rpt)])

    return k(x, src, dst, ew)


def _gather_sc(table, idx):
    n, d4 = table.shape
    m = idx.shape[0]
    per_w = m // _NW
    ch = 128
    nstep = per_w // ch

    mesh = plsc.VectorSubcoreMesh(core_axis_name="c", subcore_axis_name="s", num_cores=_NC, num_subcores=_NS)

    @functools.partial(
        pl.kernel,
        out_type=jax.ShapeDtypeStruct((m, d4), jnp.float32),
        mesh=mesh,
        scratch_types=[
            pltpu.VMEM((ch,), jnp.int32),
            pltpu.VMEM((ch, d4), jnp.float32),
            pltpu.SemaphoreType.DMA,
        ],
    )
    def k(tab_hbm, idx_hbm, out_hbm, ib, rows, sem):
        c = lax.axis_index("c")
        s = lax.axis_index("s")
        wid = s * _NC + c
        for t in range(nstep):
            base = wid * per_w + t * ch
            pltpu.sync_copy(idx_hbm.at[pl.ds(base, ch)], ib)
            pltpu.async_copy(tab_hbm.at[ib], rows, sem).wait()
            pltpu.sync_copy(rows, out_hbm.at[pl.ds(base, ch)])

    return k(table, idx)


def _proj_body(img_ref, rel_ref, att_ref, wi_ref, wr_ref, wa_ref,
               bi_ref, br_ref, ba_ref, oi_ref, or_ref, oa_ref):
    x = img_ref[...]
    nrm = jnp.sqrt(jnp.sum(x * x, axis=1, keepdims=True))
    x = x / jnp.maximum(nrm, _EPS)
    dn = (((1,), (1,)), ((), ()))
    oi_ref[...] = lax.dot_general(x, wi_ref[...], dn,
                                  preferred_element_type=jnp.float32) + bi_ref[...]
    or_ref[...] = lax.dot_general(rel_ref[...], wr_ref[...], dn,
                                  preferred_element_type=jnp.float32) + br_ref[...]
    oa_ref[...] = lax.dot_general(att_ref[...], wa_ref[...], dn,
                                  preferred_element_type=jnp.float32) + ba_ref[...]


def _proj(img, rel_p, att_p, w_img, w_rel_p, w_att_p, b_img, b_rel, b_att):
    n = img.shape[0]
    bm = 1000
    g = n // bm
    d = w_img.shape[0]
    kimg, krel = img.shape[1], rel_p.shape[1]
    out = jax.ShapeDtypeStruct((n, d), jnp.float32)
    full = lambda shp: pl.BlockSpec(shp, lambda i: tuple(0 for _ in shp))
    return pl.pallas_call(
        _proj_body,
        grid=(g,),
        in_specs=[
            pl.BlockSpec((bm, kimg), lambda i: (i, 0)),
            pl.BlockSpec((bm, krel), lambda i: (i, 0)),
            pl.BlockSpec((bm, krel), lambda i: (i, 0)),
            full((d, kimg)), full((d, krel)), full((d, krel)),
            full((d,)), full((d,)), full((d,)),
        ],
        out_specs=[pl.BlockSpec((bm, d), lambda i: (i, 0))] * 3,
        out_shape=[out, out, out],
    )(img, rel_p, att_p, w_img, w_rel_p, w_att_p, b_img, b_rel, b_att)


def _affine2(ax2, w, b, relu):
    twon, d = ax2.shape
    n = twon // _NC
    bm = 1024
    g = n // bm

    def body(a_ref, b2_ref, w_ref, bias_ref, o_ref):
        p = a_ref[...] + b2_ref[...]
        z = jnp.dot(p, w_ref[...], preferred_element_type=jnp.float32) + bias_ref[...]
        o_ref[...] = jnp.maximum(z, 0.0) if relu else z

    return pl.pallas_call(
        body,
        grid=(g,),
        in_specs=[
            pl.BlockSpec((bm, d), lambda i: (i, 0)),
            pl.BlockSpec((bm, d), lambda i: (g + i, 0)),
            pl.BlockSpec((d, d), lambda i: (0, 0)),
            pl.BlockSpec((d,), lambda i: (0,)),
        ],
        out_specs=pl.BlockSpec((bm, d), lambda i: (i, 0)),
        out_shape=jax.ShapeDtypeStruct((n, d), jnp.float32),
    )(ax2, ax2, w, b)


def _joint(img_emb, att_emb, rel_emb, gph_emb, wn):
    n, d = img_emb.shape
    bm = 1000
    g = n // bm

    def body(wn_ref, i_ref, a_ref, r_ref, g_ref, o_ref):
        for k, ref in enumerate((i_ref, a_ref, r_ref, g_ref)):
            x = ref[...]
            nrm = jnp.sqrt(jnp.sum(x * x, axis=1, keepdims=True))
            o_ref[:, k * d:(k + 1) * d] = wn_ref[k] * (x / jnp.maximum(nrm, _EPS))

    return pl.pallas_call(
        body,
        grid=(g,),
        in_specs=[
            pl.BlockSpec(memory_space=pltpu.SMEM),
        ] + [pl.BlockSpec((bm, d), lambda i: (i, 0))] * 4,
        out_specs=pl.BlockSpec((bm, 4 * d), lambda i: (i, 0)),
        out_shape=jax.ShapeDtypeStruct((n, 4 * d), jnp.float32),
    )(wn, img_emb, att_emb, rel_emb, gph_emb)


_ALPHAS = (5.0, 15.0, 15.0, 15.0, 15.0)
_GRP = (3, 2, 1, 0)


def _scores_body(scal_ref, a_ref, b_ref, *outs):
    bm = a_ref.shape[0]
    d = a_ref.shape[1] // 4
    a = a_ref[...]
    b = b_ref[...]
    dn = (((1,), (1,)), ((), ()))
    gmats = [lax.dot_general(a[:, k * d:(k + 1) * d], b[:, k * d:(k + 1) * d],
                             dn, preferred_element_type=jnp.float32)
             for k in range(4)]
    sjoint = (gmats[0] + gmats[1] + gmats[2] + gmats[3])
    args = [(gmats[_GRP[l]], _ALPHAS[l] * scal_ref[_GRP[l]]) for l in range(4)]
    args.append((sjoint, _ALPHAS[4] * scal_ref[4]))

    ones_c = jnp.ones((bm, 1), jnp.float32)
    ones_r = jnp.ones((1, bm), jnp.float32)
    j = pl.program_id(1)

    rs_refs = outs[:5]
    cp_refs = outs[5:]
    for l in range(5):
        g, sc = args[l]
        el = jnp.exp(g * sc)
        rsum = jnp.dot(el, ones_c,
                       preferred_element_type=jnp.float32).reshape(bm)
        csum = jnp.dot(ones_r, el,
                       preferred_element_type=jnp.float32).reshape(bm)

        @pl.when(j == 0)
        def _(rref=rs_refs[l], v=rsum):
            rref[...] = v

        @pl.when(j != 0)
        def _(rref=rs_refs[l], v=rsum):
            rref[...] = rref[...] + v

        cp_refs[l][...] = csum.reshape(1, 1, bm)


def _scores(jg, scal):
    m2, d4 = jg.shape
    m = m2 // 2
    bm = 512
    g = m // bm
    rs_shape = jax.ShapeDtypeStruct((m,), jnp.float32)
    cp_shape = jax.ShapeDtypeStruct((g, 1, m), jnp.float32)
    return pl.pallas_call(
        _scores_body,
        grid=(g, g),
        in_specs=[
            pl.BlockSpec(memory_space=pltpu.SMEM),
            pl.BlockSpec((bm, d4), lambda i, j: (i, 0)),
            pl.BlockSpec((bm, d4), lambda i, j: (g + j, 0)),
        ],
        out_specs=([pl.BlockSpec((bm,), lambda i, j: (i,))] * 5
                   + [pl.BlockSpec((1, 1, bm), lambda i, j: (i, 0, j))] * 5),
        out_shape=[rs_shape] * 5 + [cp_shape] * 5,
    )(scal, jg, jg)


def _finish_body(scal_ref, ai_ref, as_ref, *refs):
    i = pl.program_id(0)
    rs_refs = refs[:5]
    cp_refs = refs[5:10]
    o_ref = refs[10]
    bm = ai_ref.shape[0]
    d = ai_ref.shape[1] // 4
    ai = ai_ref[...]
    asv = as_ref[...]
    gd = [jnp.sum(ai[:, k * d:(k + 1) * d] * asv[:, k * d:(k + 1) * d], axis=1)
          for k in range(4)]
    djoint = (gd[0] + gd[1] + gd[2] + gd[3]) * scal_ref[4]
    dvals = [gd[_GRP[l]] * scal_ref[_GRP[l]] for l in range(4)] + [djoint]

    total = jnp.zeros((), jnp.float32)
    for l in range(5):
        dg = dvals[l]
        ed = jnp.exp(_ALPHAS[l] * dg)
        rs = rs_refs[l][...] - ed
        cs = jnp.sum(cp_refs[l][...], axis=(0, 1)) - ed
        term = (jnp.log1p(cs) / _ALPHAS[l]
                + jnp.log1p(rs) / _ALPHAS[l]
                - 10.0 * jnp.log1p(jnp.maximum(dg, 0.0)))
        total = total + jnp.sum(term)

    @pl.when(i == 0)
    def _():
        o_ref[...] = jnp.zeros((1, 1), jnp.float32)

    o_ref[...] = o_ref[...] + total.reshape(1, 1) * (1.0 / (bm * pl.num_programs(0)))


def _finish(jg, rs_list, cp_list, scal):
    m2, d4 = jg.shape
    m = m2 // 2
    bm = 512
    g = m // bm
    return pl.pallas_call(
        _finish_body,
        grid=(g,),
        in_specs=(
            [pl.BlockSpec(memory_space=pltpu.SMEM),
             pl.BlockSpec((bm, d4), lambda i: (i, 0)),
             pl.BlockSpec((bm, d4), lambda i: (g + i, 0))]
            + [pl.BlockSpec((bm,), lambda i: (i,))] * 5
            + [pl.BlockSpec((g, 1, bm), lambda i: (0, 0, i))] * 5
        ),
        out_specs=pl.BlockSpec((1, 1), lambda i: (0, 0)),
        out_shape=jax.ShapeDtypeStruct((1, 1), jnp.float32),
    )(scal, jg, jg, *rs_list, *cp_list)


def kernel(batch, src, dst, edge_weight, ent_weight, img_feats, rel_feats,
           att_feats, gcn_w1, gcn_b1, gcn_w2, gcn_b2, w_img, b_img, w_rel,
           b_rel, w_att, b_att, weight_raw):
    n, d = ent_weight.shape

    img_emb, rel_emb, att_emb = _proj(img_feats, rel_feats, att_feats,
                                      w_img, w_rel, w_att,
                                      b_img, b_rel, b_att)

    npad = 10240
    src32 = src.astype(jnp.int32)
    dst32 = dst.astype(jnp.int32)
    ax = _spmm_sc(ent_weight, src32, dst32, edge_weight, npad)
    h = _affine2(ax, gcn_w1, gcn_b1, relu=True)
    ah = _spmm_sc(h, src32, dst32, edge_weight, npad)
    gph_emb = _affine2(ah, gcn_w2, gcn_b2, relu=False)[:n]

    wn = jax.nn.softmax(weight_raw)
    joint = _joint(img_emb, att_emb, rel_emb, gph_emb, wn)

    bt = jnp.transpose(batch).reshape(-1).astype(jnp.int32)
    jg = _gather_sc(joint, bt)

    w2 = wn * wn
    scal = jnp.concatenate(
        [1.0 / w2, (1.0 / jnp.sum(w2))[None], jnp.zeros((3,), jnp.float32)]
    ).astype(jnp.float32)

    outs = _scores(jg, scal)
    loss = _finish(jg, outs[:5], outs[5:], scal)
    return loss.reshape(()), joint


# speedup vs baseline: 1.6540x; 1.0875x over previous
import functools

import jax
import jax.numpy as jnp
from jax import lax
from jax.experimental import pallas as pl
from jax.experimental.pallas import tpu as pltpu
from jax.experimental.pallas import tpu_sc as plsc

_NC = 2
_NS = 16
_NW = _NC * _NS

_EPS = 1e-12


def _spmm_sc(x, src, dst, ew, npad, ch=80):
    d = x.shape[1]
    e = src.shape[0]
    epw = e // _NW
    nch = epw // ch
    npair = nch // 2
    rpt = npad // _NS

    mesh = plsc.VectorSubcoreMesh(core_axis_name="c", subcore_axis_name="s", num_cores=_NC, num_subcores=_NS)

    @functools.partial(
        pl.kernel,
        out_type=jax.ShapeDtypeStruct((_NC * npad, d), jnp.float32),
        mesh=mesh,
        scratch_types=[
            pltpu.VMEM_SHARED((npad, d), jnp.float32),
            pltpu.VMEM((epw,), jnp.int32),
            pltpu.VMEM((epw,), jnp.float32),
            pltpu.VMEM((ch,), jnp.int32),
            pltpu.VMEM((ch,), jnp.int32),
            pltpu.VMEM((ch, d), jnp.float32),
            pltpu.VMEM((ch, d), jnp.float32),
            pltpu.SemaphoreType.DMA,
            pltpu.SemaphoreType.DMA,
            pltpu.SemaphoreType.DMA,
            pltpu.SemaphoreType.DMA,
            pltpu.SemaphoreType.DMA,
            pltpu.SemaphoreType.DMA,
        ],
        compiler_params=pltpu.CompilerParams(use_tc_tiling_on_sc=True),
    )
    def k(x_hbm, src_hbm, dst_hbm, ew_hbm, out_hbm,
          acc, sidx_tab, ewv_tab, didx0, didx1, rows0, rows1,
          sg0, sg1, sd0, sd1, ss0, ss1):
        c = lax.axis_index("c")
        s = lax.axis_index("s")
        w = s * _NC + c
        ebase = w * epw
        rlo = s * rpt
        rows = (rows0, rows1)
        didx = (didx0, didx1)
        gsem = (sg0, sg1)
        dsem = (sd0, sd1)
        ssem = (ss0, ss1)

        pltpu.sync_copy(src_hbm.at[pl.ds(ebase, epw)], sidx_tab)
        pltpu.sync_copy(ew_hbm.at[pl.ds(ebase, epw)], ewv_tab)

        def zrow(r, cc):
            for j in range(d // 16):
                rows0[r, pl.ds(16 * j, 16)] = jnp.zeros((16,), jnp.float32)
            return cc

        lax.fori_loop(0, ch, zrow, 0)

        def zcp(z, cc):
            pltpu.sync_copy(rows0, acc.at[pl.ds(rlo + z * ch, ch)])
            return cc

        lax.fori_loop(0, rpt // ch, zcp, 0)
        plsc.subcore_barrier()

        def dfire(i, b):
            pltpu.async_copy(dst_hbm.at[pl.ds(ebase + i * ch, ch)], didx[b], dsem[b])

        def dwait(i, b):
            pltpu.make_async_copy(dst_hbm.at[pl.ds(ebase + i * ch, ch)],
                                  didx[b], dsem[b]).wait()

        def gfire(i, b):
            pltpu.async_copy(x_hbm.at[sidx_tab.at[pl.ds(i * ch, ch)]],
                             rows[b], gsem[b])

        def gwait(i, b):
            pltpu.make_async_copy(x_hbm.at[sidx_tab.at[pl.ds(i * ch, ch)]],
                                  rows[b], gsem[b]).wait()

        def sfire(b):
            pltpu.async_copy(rows[b], acc.at[didx[b]], ssem[b], add=True)

        def swait(b):
            pltpu.make_async_copy(rows[b], acc.at[didx[b]], ssem[b]).wait()

        def scale(i, b):
            rb = rows[b]

            def grp(g, cc):
                wv = ewv_tab[pl.ds(i * ch + g * 16, 16)]
                for lane in range(16):
                    wgt = wv[lane]
                    r = g * 16 + lane
                    for j in range(d // 16):
                        sl = pl.ds(j * 16, 16)
                        rb[r, sl] = rb[r, sl] * wgt
                return cc

            lax.fori_loop(0, ch // 16, grp, 0)
            if ch % 16:
                wv = ewv_tab[pl.ds(i * ch + ch - 16, 16)]
                for lane in range(16 - ch % 16, 16):
                    wgt = wv[lane]
                    r = ch - 16 + lane
                    for j in range(d // 16):
                        sl = pl.ds(j * 16, 16)
                        rb[r, sl] = rb[r, sl] * wgt

        dfire(0, 0)
        gfire(0, 0)

        def pair(p, cc):
            i0 = 2 * p
            i1 = i0 + 1
            dfire(i1, 1)
            gfire(i1, 1)
            gwait(i0, 0)
            scale(i0, 0)
            dwait(i0, 0)
            sfire(0)
            gwait(i1, 1)
            swait(0)

            @pl.when(p + 1 < npair)
            def _():
                dfire(i0 + 2, 0)
                gfire(i0 + 2, 0)

            scale(i1, 1)
            dwait(i1, 1)
            sfire(1)
            swait(1)
            return cc

        lax.fori_loop(0, npair, pair, 0)
        if nch % 2:
            i_last = nch - 1
            dfire(i_last, 0)
            gfire(i_last, 0)
            gwait(i_last, 0)
            scale(i_last, 0)
            dwait(i_last, 0)
            sfire(0)
            swait(0)
        plsc.subcore_barrier()
        pltpu.sync_copy(acc.at[pl.ds(rlo, rpt)],
                        out_hbm.at[pl.ds(c * npad + rlo, rpt)])

    return k(x, src, dst, ew)


def _gather_sc(table, idx):
    n, d4 = table.shape
    m = idx.shape[0]
    per_w = m // _NW
    ch = 128
    nstep = per_w // ch

    mesh = plsc.VectorSubcoreMesh(core_axis_name="c", subcore_axis_name="s", num_cores=_NC, num_subcores=_NS)

    @functools.partial(
        pl.kernel,
        out_type=jax.ShapeDtypeStruct((m, d4), jnp.float32),
        mesh=mesh,
        scratch_types=[
            pltpu.VMEM((ch,), jnp.int32),
            pltpu.VMEM((ch, d4), jnp.float32),
            pltpu.SemaphoreType.DMA,
        ],
    )
    def k(tab_hbm, idx_hbm, out_hbm, ib, rows, sem):
        c = lax.axis_index("c")
        s = lax.axis_index("s")
        wid = s * _NC + c
        for t in range(nstep):
            base = wid * per_w + t * ch
            pltpu.sync_copy(idx_hbm.at[pl.ds(base, ch)], ib)
            pltpu.async_copy(tab_hbm.at[ib], rows, sem).wait()
            pltpu.sync_copy(rows, out_hbm.at[pl.ds(base, ch)])

    return k(table, idx)


def _proj_body(img_ref, rel_ref, att_ref, wi_ref, wr_ref, wa_ref,
               bi_ref, br_ref, ba_ref, oi_ref, or_ref, oa_ref):
    x = img_ref[...]
    nrm = jnp.sqrt(jnp.sum(x * x, axis=1, keepdims=True))
    x = x / jnp.maximum(nrm, _EPS)
    dn = (((1,), (1,)), ((), ()))
    oi_ref[...] = lax.dot_general(x, wi_ref[...], dn,
                                  preferred_element_type=jnp.float32) + bi_ref[...]
    or_ref[...] = lax.dot_general(rel_ref[...], wr_ref[...], dn,
                                  preferred_element_type=jnp.float32) + br_ref[...]
    oa_ref[...] = lax.dot_general(att_ref[...], wa_ref[...], dn,
                                  preferred_element_type=jnp.float32) + ba_ref[...]


def _proj(img, rel_p, att_p, w_img, w_rel_p, w_att_p, b_img, b_rel, b_att):
    n = img.shape[0]
    bm = 1000
    g = n // bm
    d = w_img.shape[0]
    kimg, krel = img.shape[1], rel_p.shape[1]
    out = jax.ShapeDtypeStruct((n, d), jnp.float32)
    full = lambda shp: pl.BlockSpec(shp, lambda i: tuple(0 for _ in shp))
    return pl.pallas_call(
        _proj_body,
        grid=(g,),
        in_specs=[
            pl.BlockSpec((bm, kimg), lambda i: (i, 0)),
            pl.BlockSpec((bm, krel), lambda i: (i, 0)),
            pl.BlockSpec((bm, krel), lambda i: (i, 0)),
            full((d, kimg)), full((d, krel)), full((d, krel)),
            full((d,)), full((d,)), full((d,)),
        ],
        out_specs=[pl.BlockSpec((bm, d), lambda i: (i, 0))] * 3,
        out_shape=[out, out, out],
    )(img, rel_p, att_p, w_img, w_rel_p, w_att_p, b_img, b_rel, b_att)


def _affine2(ax2, w, b, relu):
    twon, d = ax2.shape
    n = twon // _NC
    bm = 1024
    g = n // bm

    def body(a_ref, b2_ref, w_ref, bias_ref, o_ref):
        p = a_ref[...] + b2_ref[...]
        z = jnp.dot(p, w_ref[...], preferred_element_type=jnp.float32) + bias_ref[...]
        o_ref[...] = jnp.maximum(z, 0.0) if relu else z

    return pl.pallas_call(
        body,
        grid=(g,),
        in_specs=[
            pl.BlockSpec((bm, d), lambda i: (i, 0)),
            pl.BlockSpec((bm, d), lambda i: (g + i, 0)),
            pl.BlockSpec((d, d), lambda i: (0, 0)),
            pl.BlockSpec((d,), lambda i: (0,)),
        ],
        out_specs=pl.BlockSpec((bm, d), lambda i: (i, 0)),
        out_shape=jax.ShapeDtypeStruct((n, d), jnp.float32),
    )(ax2, ax2, w, b)


def _joint(img_emb, att_emb, rel_emb, gph_emb, wn):
    n, d = img_emb.shape
    bm = 1000
    g = n // bm

    def body(wn_ref, i_ref, a_ref, r_ref, g_ref, o_ref):
        for k, ref in enumerate((i_ref, a_ref, r_ref, g_ref)):
            x = ref[...]
            nrm = jnp.sqrt(jnp.sum(x * x, axis=1, keepdims=True))
            o_ref[:, k * d:(k + 1) * d] = wn_ref[k] * (x / jnp.maximum(nrm, _EPS))

    return pl.pallas_call(
        body,
        grid=(g,),
        in_specs=[
            pl.BlockSpec(memory_space=pltpu.SMEM),
        ] + [pl.BlockSpec((bm, d), lambda i: (i, 0))] * 4,
        out_specs=pl.BlockSpec((bm, 4 * d), lambda i: (i, 0)),
        out_shape=jax.ShapeDtypeStruct((n, 4 * d), jnp.float32),
    )(wn, img_emb, att_emb, rel_emb, gph_emb)


_ALPHAS = (5.0, 15.0, 15.0, 15.0, 15.0)
_GRP = (3, 2, 1, 0)


def _scores_body(scal_ref, a_ref, b_ref, *outs):
    bm = a_ref.shape[0]
    d = a_ref.shape[1] // 4
    a = a_ref[...]
    b = b_ref[...]
    dn = (((1,), (1,)), ((), ()))
    gmats = [lax.dot_general(a[:, k * d:(k + 1) * d], b[:, k * d:(k + 1) * d],
                             dn, preferred_element_type=jnp.float32)
             for k in range(4)]
    sjoint = (gmats[0] + gmats[1] + gmats[2] + gmats[3])
    args = [(gmats[_GRP[l]], _ALPHAS[l] * scal_ref[_GRP[l]]) for l in range(4)]
    args.append((sjoint, _ALPHAS[4] * scal_ref[4]))

    j = pl.program_id(1)

    rs_refs = outs[:5]
    cp_refs = outs[5:]
    for l in range(5):
        g, sc = args[l]
        el = jnp.exp(g * sc)
        rsum = jnp.sum(el, axis=1)
        csum = jnp.sum(el, axis=0)

        @pl.when(j == 0)
        def _(rref=rs_refs[l], v=rsum):
            rref[...] = v

        @pl.when(j != 0)
        def _(rref=rs_refs[l], v=rsum):
            rref[...] = rref[...] + v

        cp_refs[l][...] = csum.reshape(1, 1, bm)


def _scores(jg, scal):
    m2, d4 = jg.shape
    m = m2 // 2
    bm = 512
    g = m // bm
    rs_shape = jax.ShapeDtypeStruct((m,), jnp.float32)
    cp_shape = jax.ShapeDtypeStruct((g, 1, m), jnp.float32)
    return pl.pallas_call(
        _scores_body,
        grid=(g, g),
        in_specs=[
            pl.BlockSpec(memory_space=pltpu.SMEM),
            pl.BlockSpec((bm, d4), lambda i, j: (i, 0)),
            pl.BlockSpec((bm, d4), lambda i, j: (g + j, 0)),
        ],
        out_specs=([pl.BlockSpec((bm,), lambda i, j: (i,))] * 5
                   + [pl.BlockSpec((1, 1, bm), lambda i, j: (i, 0, j))] * 5),
        out_shape=[rs_shape] * 5 + [cp_shape] * 5,
    )(scal, jg, jg)


def _finish_body(scal_ref, ai_ref, as_ref, *refs):
    i = pl.program_id(0)
    rs_refs = refs[:5]
    cp_refs = refs[5:10]
    o_ref = refs[10]
    bm = ai_ref.shape[0]
    d = ai_ref.shape[1] // 4
    ai = ai_ref[...]
    asv = as_ref[...]
    gd = [jnp.sum(ai[:, k * d:(k + 1) * d] * asv[:, k * d:(k + 1) * d], axis=1)
          for k in range(4)]
    djoint = (gd[0] + gd[1] + gd[2] + gd[3]) * scal_ref[4]
    dvals = [gd[_GRP[l]] * scal_ref[_GRP[l]] for l in range(4)] + [djoint]

    total = jnp.zeros((), jnp.float32)
    for l in range(5):
        dg = dvals[l]
        ed = jnp.exp(_ALPHAS[l] * dg)
        rs = rs_refs[l][...] - ed
        cs = jnp.sum(cp_refs[l][...], axis=(0, 1)) - ed
        term = (jnp.log1p(cs) / _ALPHAS[l]
                + jnp.log1p(rs) / _ALPHAS[l]
                - 10.0 * jnp.log1p(jnp.maximum(dg, 0.0)))
        total = total + jnp.sum(term)

    @pl.when(i == 0)
    def _():
        o_ref[...] = jnp.zeros((1, 1), jnp.float32)

    o_ref[...] = o_ref[...] + total.reshape(1, 1) * (1.0 / (bm * pl.num_programs(0)))


def _finish(jg, rs_list, cp_list, scal):
    m2, d4 = jg.shape
    m = m2 // 2
    bm = 512
    g = m // bm
    return pl.pallas_call(
        _finish_body,
        grid=(g,),
        in_specs=(
            [pl.BlockSpec(memory_space=pltpu.SMEM),
             pl.BlockSpec((bm, d4), lambda i: (i, 0)),
             pl.BlockSpec((bm, d4), lambda i: (g + i, 0))]
            + [pl.BlockSpec((bm,), lambda i: (i,))] * 5
            + [pl.BlockSpec((g, 1, bm), lambda i: (0, 0, i))] * 5
        ),
        out_specs=pl.BlockSpec((1, 1), lambda i: (0, 0)),
        out_shape=jax.ShapeDtypeStruct((1, 1), jnp.float32),
    )(scal, jg, jg, *rs_list, *cp_list)


def kernel(batch, src, dst, edge_weight, ent_weight, img_feats, rel_feats,
           att_feats, gcn_w1, gcn_b1, gcn_w2, gcn_b2, w_img, b_img, w_rel,
           b_rel, w_att, b_att, weight_raw):
    n, d = ent_weight.shape

    img_emb, rel_emb, att_emb = _proj(img_feats, rel_feats, att_feats,
                                      w_img, w_rel, w_att,
                                      b_img, b_rel, b_att)

    npad = 10240
    src32 = src.astype(jnp.int32)
    dst32 = dst.astype(jnp.int32)
    ax = _spmm_sc(ent_weight, src32, dst32, edge_weight, npad)
    h = _affine2(ax, gcn_w1, gcn_b1, relu=True)
    ah = _spmm_sc(h, src32, dst32, edge_weight, npad)
    gph_emb = _affine2(ah, gcn_w2, gcn_b2, relu=False)[:n]

    wn = jax.nn.softmax(weight_raw)
    joint = _joint(img_emb, att_emb, rel_emb, gph_emb, wn)

    bt = jnp.transpose(batch).reshape(-1).astype(jnp.int32)
    jg = _gather_sc(joint, bt)

    w2 = wn * wn
    scal = jnp.concatenate(
        [1.0 / w2, (1.0 / jnp.sum(w2))[None], jnp.zeros((3,), jnp.float32)]
    ).astype(jnp.float32)

    outs = _scores(jg, scal)
    loss = _finish(jg, outs[:5], outs[5:], scal)
    return loss.reshape(()), joint
